# Initial kernel scaffold; baseline (speedup 1.0000x reference)
import jax
import jax.numpy as jnp
from jax.experimental import pallas as pl


def kernel(tokenized_prompts, token_embedding):
    raise NotImplementedError("write your pallas kernel here")


# speedup vs baseline: 1.2764x; 1.2764x over previous
import functools

import jax
import jax.numpy as jnp
from jax import lax
from jax.experimental import pallas as pl
from jax.experimental.pallas import tpu as pltpu
from jax.experimental.pallas import tpu_sc as plsc

N_CLS = 1024
CTX_LEN = 77
VOCAB = 49408
CTX_DIM = 512

B = N_CLS * CTX_LEN
NW = 32
ROWS_PER_W = B // NW
CHUNK = 112
NCHUNK = ROWS_PER_W // CHUNK

_mesh = plsc.VectorSubcoreMesh(core_axis_name="c", subcore_axis_name="s")


@functools.partial(
    pl.kernel,
    out_type=jax.ShapeDtypeStruct((B, CTX_DIM), jnp.float32),
    mesh=_mesh,
    scratch_types=[
        pltpu.VMEM((ROWS_PER_W,), jnp.int32),
        pltpu.VMEM((CHUNK, CTX_DIM), jnp.float32),
        pltpu.VMEM((CHUNK, CTX_DIM), jnp.float32),
        pltpu.SemaphoreType.DMA,
        pltpu.SemaphoreType.DMA,
    ],
)
def _gather(idx_hbm, table_hbm, out_hbm, idx_v, buf0, buf1, sem0, sem1):
    wid = lax.axis_index("s") * 2 + lax.axis_index("c")
    base = pl.multiple_of(wid * ROWS_PER_W, ROWS_PER_W)

    pltpu.sync_copy(idx_hbm.at[pl.ds(base, ROWS_PER_W)], idx_v)

    def start_gather(j, buf, sem):
        off = pl.multiple_of(j * CHUNK, CHUNK)
        return pltpu.async_copy(table_hbm.at[idx_v.at[pl.ds(off, CHUNK)]], buf, sem)

    def wait_gather(buf, sem):
        pltpu.make_async_copy(table_hbm.at[idx_v.at[pl.ds(0, CHUNK)]], buf, sem).wait()

    def store(j, buf):
        off = pl.multiple_of(base + j * CHUNK, CHUNK)
        pltpu.sync_copy(buf, out_hbm.at[pl.ds(off, CHUNK)])

    start_gather(0, buf0, sem0)
    start_gather(1, buf1, sem1)

    @pl.loop(0, (NCHUNK - 2) // 2)
    def _steady(i):
        j = i * 2
        wait_gather(buf0, sem0)
        store(j, buf0)
        start_gather(j + 2, buf0, sem0)
        wait_gather(buf1, sem1)
        store(j + 1, buf1)
        start_gather(j + 3, buf1, sem1)

    wait_gather(buf0, sem0)
    store(NCHUNK - 2, buf0)
    wait_gather(buf1, sem1)
    store(NCHUNK - 1, buf1)


def kernel(tokenized_prompts, token_embedding):
    idx = tokenized_prompts.reshape(B)
    out = _gather(idx, token_embedding)
    return out.reshape(N_CLS, CTX_LEN, CTX_DIM)
